# baseline (speedup 1.0000x reference)
import functools

import jax
import jax.numpy as jnp
from jax import lax
from jax.experimental import pallas as pl
from jax.experimental.pallas import tpu as pltpu
from jax.experimental.pallas import tpu_sc as plsc

N = 10000
NP = 10240
R = 512
NB = NP // R
BLK = 128
DEPTH = 2
NSUB = 16
NCORE = 2
RPT = NP // NSUB

F0 = 128
F1, C1 = 128, 8
F2, C2 = 128, 16


def _mesh():
    return plsc.VectorSubcoreMesh(core_axis_name="c", subcore_axis_name="s")


def _make_deg_kernel(e_pad):
    ept = e_pad // (NCORE * NSUB)

    @functools.partial(
        pl.kernel,
        out_type=jax.ShapeDtypeStruct((NCORE * NSUB, NP), jnp.float32),
        mesh=_mesh(),
        compiler_params=pltpu.CompilerParams(needs_layout_passes=False),
        scratch_types=[
            pltpu.VMEM((ept,), jnp.int32),
            pltpu.VMEM((NP,), jnp.float32),
        ],
    )
    def deg_kernel(dst_hbm, hist_hbm, dstb, hist):
        core = lax.axis_index("c")
        sub = lax.axis_index("s")
        wid = sub * NCORE + core

        zeros16 = jnp.zeros((16,), jnp.float32)

        def zbody(i, carry):
            hist[pl.ds(i * 16, 16)] = zeros16
            return carry

        lax.fori_loop(0, NP // 16, zbody, 0)

        pltpu.sync_copy(dst_hbm.at[pl.ds(wid * ept, ept)], dstb)
        ones16 = jnp.ones((16,), jnp.float32)

        def body(i, carry):
            d16 = dstb[pl.ds(i * 16, 16)]
            plsc.addupdate_scatter(hist, [d16], ones16)
            return carry

        lax.fori_loop(0, ept // 16, body, 0)
        pltpu.sync_copy(hist, hist_hbm.at[wid])

    return deg_kernel


def _make_prop_kernel(e_pad, n_chunks, feat, split_edges=False):
    n_out = 2 if split_edges else n_chunks
    nblk_t = e_pad // (NSUB * (NCORE if split_edges else 1) * BLK)
    nstage = nblk_t // 8

    @functools.partial(
        pl.kernel,
        out_type=jax.ShapeDtypeStruct((n_out * NP, feat), jnp.float32),
        mesh=_mesh(),
        compiler_params=pltpu.CompilerParams(needs_layout_passes=False),
        scratch_types=[
            pltpu.VMEM((nstage, 1, BLK), jnp.int32),
            pltpu.VMEM((nstage, 1, BLK), jnp.int32),
        ] + [pltpu.VMEM((BLK,), jnp.int32)] * DEPTH
          + [pltpu.VMEM((BLK,), jnp.int32)]
          + [pltpu.VMEM((BLK, feat), jnp.float32)] * DEPTH
          + [pltpu.VMEM_SHARED((NP, feat), jnp.float32)]
          + [pltpu.SemaphoreType.DMA] * DEPTH,
    )
    def prop_kernel(z_hbm, src_hbm, dst_hbm, zero_hbm, y_hbm, srcS, dstS,
                    *bufs):
        gbufs = bufs[0:DEPTH]
        dbuf = bufs[DEPTH]
        rowss = bufs[DEPTH + 1:2 * DEPTH + 1]
        acc = bufs[2 * DEPTH + 1]
        gsems = bufs[2 * DEPTH + 2:]
        core = lax.axis_index("c")
        sub = lax.axis_index("s")

        def run_part(c, blk_base, with_offset):
            pltpu.sync_copy(src_hbm.at[pl.ds(blk_base, nstage)], srcS)
            pltpu.sync_copy(dst_hbm.at[pl.ds(blk_base, nstage)], dstS)

            def fire(g, i):
                gbuf = gbufs[i]
                for k in range(BLK // 16):
                    s16 = srcS[g, 0, pl.ds(k * 16, 16)]
                    gbuf[pl.ds(k * 16, 16)] = (
                        s16 + c * NP if with_offset else s16)
                pltpu.async_copy(z_hbm.at[gbuf], rowss[i], gsems[i])

            def wait_scatter(g, i):
                for k in range(BLK // 16):
                    dbuf[pl.ds(k * 16, 16)] = dstS[g, 0, pl.ds(k * 16, 16)]
                pltpu.make_async_copy(
                    z_hbm.at[gbufs[i]], rowss[i], gsems[i]).wait()
                pltpu.sync_copy(rowss[i], acc.at[dbuf], add=True)

            for i in range(DEPTH - 1):
                fire(i, i)

            def step(q, carry):
                a = DEPTH * q
                fire(a + DEPTH - 1, DEPTH - 1)
                for i in range(DEPTH):
                    wait_scatter(a + i, i)
                    if i < DEPTH - 1:
                        @pl.when(a + DEPTH + i < nstage)
                        def _(i=i):
                            fire(a + DEPTH + i, i)
                return carry

            lax.fori_loop(0, nstage // DEPTH, step, 0)

        def run_chunk(c, blk_base, out_base, with_offset):
            pltpu.sync_copy(zero_hbm, acc.at[pl.ds(sub * RPT, RPT)])
            plsc.subcore_barrier()

            def pbody(part, carry):
                run_part(c, blk_base + part * nstage, with_offset)
                return carry

            lax.fori_loop(0, 8, pbody, 0)
            plsc.subcore_barrier()
            pltpu.sync_copy(
                acc.at[pl.ds(sub * RPT, RPT)],
                y_hbm.at[pl.ds(pl.multiple_of(out_base + sub * RPT, 8), RPT)])
            plsc.subcore_barrier()

        if split_edges:
            run_chunk(0, (core * NSUB + sub) * nblk_t, core * NP, False)
        else:
            def cbody(j, carry):
                c = 2 * j + core
                run_chunk(c, sub * nblk_t, c * NP, True)
                return carry

            lax.fori_loop(0, n_chunks // NCORE, cbody, 0)

    return prop_kernel


def _row_mask(r_id, x):
    rows = lax.broadcasted_iota(jnp.int32, x.shape, 0) + r_id * R
    return jnp.where(rows < N, x, 0.0)


def _tc0_body(hist_ref, v_ref, z0_ref, deg_ref):
    deg = jnp.sum(hist_ref[...], axis=0) + 1.0
    dis = lax.rsqrt(deg)[:, None]
    z0_ref[...] = v_ref[...] * dis
    deg_ref[...] = deg[:, None]


def _tc1_body(y00, y01, z00, deg_ref, w_ref, b_ref, z1_ref):
    r = pl.program_id(0)
    dis = lax.rsqrt(deg_ref[...])
    t = (y00[...] + y01[...] + z00[...]) * dis
    x = jnp.dot(t.astype(jnp.bfloat16), w_ref[...].astype(jnp.bfloat16),
                preferred_element_type=jnp.float32)
    x = jnp.maximum(x + b_ref[0], 0.0)
    z1_ref[...] = _row_mask(r, x) * dis


def _tc2_body(y_ref, z_ref, deg_ref, w_ref, b_ref, z2_ref, acc):
    r = pl.program_id(0)
    ci = pl.program_id(2)

    @pl.when(ci == 0)
    def _():
        acc[...] = jnp.zeros_like(acc)

    dis = lax.rsqrt(deg_ref[...])
    t = (y_ref[...] + z_ref[...]) * dis
    acc[...] += jnp.dot(t.astype(jnp.bfloat16), w_ref[...].astype(jnp.bfloat16),
                        preferred_element_type=jnp.float32)

    @pl.when(ci == C1 - 1)
    def _():
        x = jnp.maximum(acc[...] + b_ref[0], 0.0)
        z2_ref[...] = _row_mask(r, x) * dis


def _tc3_body(y_ref, z_ref, deg_ref, wmu_ref, wstd_ref, bmu_ref, bstd_ref,
              mu_ref, std_ref, accm, accs):
    ci = pl.program_id(1)

    @pl.when(ci == 0)
    def _():
        accm[...] = jnp.zeros_like(accm)
        accs[...] = jnp.zeros_like(accs)

    dis = lax.rsqrt(deg_ref[...])
    t = ((y_ref[...] + z_ref[...]) * dis).astype(jnp.bfloat16)
    accm[...] += jnp.dot(t, wmu_ref[...].astype(jnp.bfloat16),
                         preferred_element_type=jnp.float32)
    accs[...] += jnp.dot(t, wstd_ref[...].astype(jnp.bfloat16),
                         preferred_element_type=jnp.float32)

    @pl.when(ci == C2 - 1)
    def _():
        mu_ref[...] = accm[...] + bmu_ref[...]
        std_ref[...] = accs[...] + bstd_ref[...]


def kernel(v, edge_index, W1, b1, W2, b2, Wmu, bmu, Wstd, bstd):
    f32 = jnp.float32
    e = edge_index.shape[1]
    quantum = NSUB * NCORE * BLK * 8
    e_pad = -(-e // quantum) * quantum

    src = jnp.pad(edge_index[0], (0, e_pad - e), constant_values=N)
    dst = jnp.pad(edge_index[1], (0, e_pad - e), constant_values=N)
    src2 = src.reshape(-1, 1, BLK)
    dst2 = dst.reshape(-1, 1, BLK)
    v_pad = jnp.pad(v, ((0, NP - N), (0, 0)))
    W2p = jnp.pad(W2, ((0, 0), (0, C2 * F2 - W2.shape[1])))
    b2p = jnp.pad(b2, (0, C2 * F2 - b2.shape[0]))
    Wmu_p = jnp.pad(Wmu, ((0, C2 * F2 - Wmu.shape[0]), (0, 0)))
    Wstd_p = jnp.pad(Wstd, ((0, C2 * F2 - Wstd.shape[0]), (0, 0)))
    b1r = b1.reshape(C1, 1, F1)
    b2r = b2p.reshape(C2, 1, F2)

    hist = _make_deg_kernel(e_pad)(dst)

    z0, deg_col = pl.pallas_call(
        _tc0_body,
        grid=(NB,),
        in_specs=[
            pl.BlockSpec((NCORE * NSUB, R), lambda r: (0, r)),
            pl.BlockSpec((R, F0), lambda r: (r, 0)),
        ],
        out_specs=[
            pl.BlockSpec((R, F0), lambda r: (r, 0)),
            pl.BlockSpec((R, 1), lambda r: (r, 0)),
        ],
        out_shape=[
            jax.ShapeDtypeStruct((NP, F0), f32),
            jax.ShapeDtypeStruct((NP, 1), f32),
        ],
    )(hist, v_pad)

    zeros0 = jnp.zeros((RPT, F0), f32)
    y0 = _make_prop_kernel(e_pad, 1, F0, split_edges=True)(
        z0, src2, dst2, zeros0)

    z1 = pl.pallas_call(
        _tc1_body,
        grid=(NB, C1),
        in_specs=[
            pl.BlockSpec((R, F0), lambda r, co: (r, 0)),
            pl.BlockSpec((R, F0), lambda r, co: (NB + r, 0)),
            pl.BlockSpec((R, F0), lambda r, co: (r, 0)),
            pl.BlockSpec((R, 1), lambda r, co: (r, 0)),
            pl.BlockSpec((F0, F1), lambda r, co: (0, co)),
            pl.BlockSpec((1, 1, F1), lambda r, co: (co, 0, 0)),
        ],
        out_specs=pl.BlockSpec((R, F1), lambda r, co: (co * NB + r, 0)),
        out_shape=jax.ShapeDtypeStruct((C1 * NP, F1), f32),
    )(y0, y0, z0, deg_col, W1, b1r)

    zeros1 = jnp.zeros((RPT, F1), f32)
    y1 = _make_prop_kernel(e_pad, C1, F1)(z1, src2, dst2, zeros1)

    z2 = pl.pallas_call(
        _tc2_body,
        grid=(NB, C2, C1),
        in_specs=[
            pl.BlockSpec((R, F1), lambda r, co, ci: (ci * NB + r, 0)),
            pl.BlockSpec((R, F1), lambda r, co, ci: (ci * NB + r, 0)),
            pl.BlockSpec((R, 1), lambda r, co, ci: (r, 0)),
            pl.BlockSpec((F1, F2), lambda r, co, ci: (ci, co)),
            pl.BlockSpec((1, 1, F2), lambda r, co, ci: (co, 0, 0)),
        ],
        out_specs=pl.BlockSpec((R, F2), lambda r, co, ci: (co * NB + r, 0)),
        out_shape=jax.ShapeDtypeStruct((C2 * NP, F2), f32),
        scratch_shapes=[pltpu.VMEM((R, F2), f32)],
    )(y1, z1, deg_col, W2p, b2r)

    zeros2 = jnp.zeros((RPT, F2), f32)
    y2 = _make_prop_kernel(e_pad, C2, F2)(z2, src2, dst2, zeros2)

    dout = Wmu.shape[1]
    mu_pad, std_pad = pl.pallas_call(
        _tc3_body,
        grid=(NB, C2),
        in_specs=[
            pl.BlockSpec((R, F2), lambda r, ci: (ci * NB + r, 0)),
            pl.BlockSpec((R, F2), lambda r, ci: (ci * NB + r, 0)),
            pl.BlockSpec((R, 1), lambda r, ci: (r, 0)),
            pl.BlockSpec((F2, dout), lambda r, ci: (ci, 0)),
            pl.BlockSpec((F2, dout), lambda r, ci: (ci, 0)),
            pl.BlockSpec((1, dout), lambda r, ci: (0, 0)),
            pl.BlockSpec((1, dout), lambda r, ci: (0, 0)),
        ],
        out_specs=[
            pl.BlockSpec((R, dout), lambda r, ci: (r, 0)),
            pl.BlockSpec((R, dout), lambda r, ci: (r, 0)),
        ],
        out_shape=[
            jax.ShapeDtypeStruct((NP, dout), f32),
            jax.ShapeDtypeStruct((NP, dout), f32),
        ],
        scratch_shapes=[pltpu.VMEM((R, dout), f32), pltpu.VMEM((R, dout), f32)],
    )(y2, z2, deg_col, Wmu_p, Wstd_p, bmu.reshape(1, dout),
      bstd.reshape(1, dout))

    return (mu_pad[:N], std_pad[:N])


# speedup vs baseline: 1.1748x; 1.1748x over previous
import functools

import jax
import jax.numpy as jnp
from jax import lax
from jax.experimental import pallas as pl
from jax.experimental.pallas import tpu as pltpu
from jax.experimental.pallas import tpu_sc as plsc

N = 10000
NP = 10240
R = 512
NB = NP // R
BLK = 128
DEPTH = 2
NSUB = 16
NCORE = 2
RPT = NP // NSUB

F0 = 128
F1, C1 = 128, 8
F2, C2 = 128, 16


def _mesh():
    return plsc.VectorSubcoreMesh(core_axis_name="c", subcore_axis_name="s")


def _make_deg_kernel(e_pad):
    ept = e_pad // (NCORE * NSUB)

    @functools.partial(
        pl.kernel,
        out_type=jax.ShapeDtypeStruct((NCORE * NSUB, NP), jnp.float32),
        mesh=_mesh(),
        compiler_params=pltpu.CompilerParams(needs_layout_passes=False),
        scratch_types=[
            pltpu.VMEM((ept,), jnp.int32),
            pltpu.VMEM((NP,), jnp.float32),
        ],
    )
    def deg_kernel(dst_hbm, hist_hbm, dstb, hist):
        core = lax.axis_index("c")
        sub = lax.axis_index("s")
        wid = sub * NCORE + core

        zeros16 = jnp.zeros((16,), jnp.float32)

        def zbody(i, carry):
            hist[pl.ds(i * 16, 16)] = zeros16
            return carry

        lax.fori_loop(0, NP // 16, zbody, 0)

        pltpu.sync_copy(dst_hbm.at[pl.ds(wid * ept, ept)], dstb)
        ones16 = jnp.ones((16,), jnp.float32)

        def body(i, carry):
            d16 = dstb[pl.ds(i * 16, 16)]
            plsc.addupdate_scatter(hist, [d16], ones16)
            return carry

        lax.fori_loop(0, ept // 16, body, 0)
        pltpu.sync_copy(hist, hist_hbm.at[wid])

    return deg_kernel


def _make_prop_kernel(e_pad, n_chunks, feat, split_edges=False):
    n_out = 2 if split_edges else n_chunks
    nblk_t = e_pad // (NSUB * (NCORE if split_edges else 1) * BLK)
    nstage = nblk_t // 4

    @functools.partial(
        pl.kernel,
        out_type=jax.ShapeDtypeStruct((n_out * NP, feat), jnp.float32),
        mesh=_mesh(),
        compiler_params=pltpu.CompilerParams(needs_layout_passes=False),
        scratch_types=[
            pltpu.VMEM((nstage, 1, BLK), jnp.int32),
            pltpu.VMEM((nstage, 1, BLK), jnp.int32),
        ] + [pltpu.VMEM((BLK,), jnp.int32)] * DEPTH
          + [pltpu.VMEM((BLK,), jnp.int32)]
          + [pltpu.VMEM((BLK, feat), jnp.float32)] * DEPTH
          + [pltpu.VMEM_SHARED((NP, feat), jnp.float32)]
          + [pltpu.SemaphoreType.DMA] * DEPTH,
    )
    def prop_kernel(z_hbm, src_hbm, dst_hbm, zero_hbm, y_hbm, srcS, dstS,
                    *bufs):
        gbufs = bufs[0:DEPTH]
        dbuf = bufs[DEPTH]
        rowss = bufs[DEPTH + 1:2 * DEPTH + 1]
        acc = bufs[2 * DEPTH + 1]
        gsems = bufs[2 * DEPTH + 2:]
        core = lax.axis_index("c")
        sub = lax.axis_index("s")

        def run_part(c, blk_base, with_offset):
            pltpu.sync_copy(src_hbm.at[pl.ds(blk_base, nstage)], srcS)
            pltpu.sync_copy(dst_hbm.at[pl.ds(blk_base, nstage)], dstS)

            def fire(g, i):
                gbuf = gbufs[i]
                for k in range(BLK // 16):
                    s16 = srcS[g, 0, pl.ds(k * 16, 16)]
                    gbuf[pl.ds(k * 16, 16)] = (
                        s16 + c * NP if with_offset else s16)
                pltpu.async_copy(z_hbm.at[gbuf], rowss[i], gsems[i])

            def wait_scatter(g, i):
                for k in range(BLK // 16):
                    dbuf[pl.ds(k * 16, 16)] = dstS[g, 0, pl.ds(k * 16, 16)]
                pltpu.make_async_copy(
                    z_hbm.at[gbufs[i]], rowss[i], gsems[i]).wait()
                pltpu.sync_copy(rowss[i], acc.at[dbuf], add=True)

            for i in range(DEPTH - 1):
                fire(i, i)

            def step(q, carry):
                a = DEPTH * q
                fire(a + DEPTH - 1, DEPTH - 1)
                for i in range(DEPTH):
                    wait_scatter(a + i, i)
                    if i < DEPTH - 1:
                        @pl.when(a + DEPTH + i < nstage)
                        def _(i=i):
                            fire(a + DEPTH + i, i)
                return carry

            lax.fori_loop(0, nstage // DEPTH, step, 0)

        def run_chunk(c, blk_base, out_base, with_offset):
            pltpu.sync_copy(zero_hbm, acc.at[pl.ds(sub * RPT, RPT)])
            plsc.subcore_barrier()

            def pbody(part, carry):
                run_part(c, blk_base + part * nstage, with_offset)
                return carry

            lax.fori_loop(0, 4, pbody, 0)
            plsc.subcore_barrier()
            pltpu.sync_copy(
                acc.at[pl.ds(sub * RPT, RPT)],
                y_hbm.at[pl.ds(pl.multiple_of(out_base + sub * RPT, 8), RPT)])
            plsc.subcore_barrier()

        if split_edges:
            run_chunk(0, (core * NSUB + sub) * nblk_t, core * NP, False)
        else:
            def cbody(j, carry):
                c = 2 * j + core
                run_chunk(c, sub * nblk_t, c * NP, True)
                return carry

            lax.fori_loop(0, n_chunks // NCORE, cbody, 0)

    return prop_kernel


def _row_mask(r_id, x):
    rows = lax.broadcasted_iota(jnp.int32, x.shape, 0) + r_id * R
    return jnp.where(rows < N, x, 0.0)


def _tc0_body(hist_ref, v_ref, z0_ref, deg_ref):
    deg = jnp.sum(hist_ref[...], axis=0) + 1.0
    dis = lax.rsqrt(deg)[:, None]
    z0_ref[...] = v_ref[...] * dis
    deg_ref[...] = deg[:, None]


def _tc1_body(y00, y01, z00, deg_ref, w_ref, b_ref, z1_ref):
    r = pl.program_id(0)
    dis = lax.rsqrt(deg_ref[...])
    t = (y00[...] + y01[...] + z00[...]) * dis
    x = jnp.dot(t.astype(jnp.bfloat16), w_ref[...],
                preferred_element_type=jnp.float32)
    x = jnp.maximum(x + b_ref[0], 0.0)
    z1_ref[...] = _row_mask(r, x) * dis


def _tc2a_body(y_ref, z_ref, deg_ref, w_ref, x_ref, acc):
    ci = pl.program_id(1)

    @pl.when(ci == 0)
    def _():
        acc[...] = jnp.zeros_like(acc)

    dis = lax.rsqrt(deg_ref[...])
    t = ((y_ref[...] + z_ref[...]) * dis).astype(jnp.bfloat16)
    acc[...] += jnp.dot(t, w_ref[...], preferred_element_type=jnp.float32)

    @pl.when(ci == C1 - 1)
    def _():
        x_ref[...] = acc[...]


def _tc2b_body(x_ref, deg_ref, b_ref, z2_ref):
    r = pl.program_id(0)
    dis = lax.rsqrt(deg_ref[...])
    x = jnp.maximum(x_ref[...] + b_ref[0], 0.0)
    z2_ref[...] = _row_mask(r, x) * dis


def _tc3_body(y_ref, z_ref, deg_ref, wmu_ref, wstd_ref, bmu_ref, bstd_ref,
              mu_ref, std_ref, accm, accs):
    ci = pl.program_id(1)

    @pl.when(ci == 0)
    def _():
        accm[...] = jnp.zeros_like(accm)
        accs[...] = jnp.zeros_like(accs)

    dis = lax.rsqrt(deg_ref[...])
    t = ((y_ref[...] + z_ref[...]) * dis).astype(jnp.bfloat16)
    accm[...] += jnp.dot(t, wmu_ref[...], preferred_element_type=jnp.float32)
    accs[...] += jnp.dot(t, wstd_ref[...], preferred_element_type=jnp.float32)

    @pl.when(ci == C2 - 1)
    def _():
        mu_ref[...] = accm[...] + bmu_ref[...]
        std_ref[...] = accs[...] + bstd_ref[...]


def kernel(v, edge_index, W1, b1, W2, b2, Wmu, bmu, Wstd, bstd):
    f32 = jnp.float32
    e = edge_index.shape[1]
    quantum = NSUB * NCORE * BLK * 8
    e_pad = -(-e // quantum) * quantum

    src = jnp.pad(edge_index[0], (0, e_pad - e), constant_values=N)
    dst = jnp.pad(edge_index[1], (0, e_pad - e), constant_values=N)
    src2 = src.reshape(-1, 1, BLK)
    dst2 = dst.reshape(-1, 1, BLK)
    v_pad = jnp.pad(v, ((0, NP - N), (0, 0)))
    W2p = jnp.pad(W2, ((0, 0), (0, C2 * F2 - W2.shape[1]))).astype(jnp.bfloat16)
    b2p = jnp.pad(b2, (0, C2 * F2 - b2.shape[0]))
    Wmu_p = jnp.pad(Wmu, ((0, C2 * F2 - Wmu.shape[0]), (0, 0))).astype(
        jnp.bfloat16)
    Wstd_p = jnp.pad(Wstd, ((0, C2 * F2 - Wstd.shape[0]), (0, 0))).astype(
        jnp.bfloat16)
    W1b = W1.astype(jnp.bfloat16)
    b1r = b1.reshape(C1, 1, F1)
    b2r = b2p.reshape(C2, 1, F2)

    hist = _make_deg_kernel(e_pad)(dst)

    z0, deg_col = pl.pallas_call(
        _tc0_body,
        grid=(NB,),
        in_specs=[
            pl.BlockSpec((NCORE * NSUB, R), lambda r: (0, r)),
            pl.BlockSpec((R, F0), lambda r: (r, 0)),
        ],
        out_specs=[
            pl.BlockSpec((R, F0), lambda r: (r, 0)),
            pl.BlockSpec((R, 1), lambda r: (r, 0)),
        ],
        out_shape=[
            jax.ShapeDtypeStruct((NP, F0), f32),
            jax.ShapeDtypeStruct((NP, 1), f32),
        ],
    )(hist, v_pad)

    zeros0 = jnp.zeros((RPT, F0), f32)
    y0 = _make_prop_kernel(e_pad, 1, F0, split_edges=True)(
        z0, src2, dst2, zeros0)

    z1 = pl.pallas_call(
        _tc1_body,
        grid=(NB, C1),
        in_specs=[
            pl.BlockSpec((R, F0), lambda r, co: (r, 0)),
            pl.BlockSpec((R, F0), lambda r, co: (NB + r, 0)),
            pl.BlockSpec((R, F0), lambda r, co: (r, 0)),
            pl.BlockSpec((R, 1), lambda r, co: (r, 0)),
            pl.BlockSpec((F0, F1), lambda r, co: (0, co)),
            pl.BlockSpec((1, 1, F1), lambda r, co: (co, 0, 0)),
        ],
        out_specs=pl.BlockSpec((R, F1), lambda r, co: (co * NB + r, 0)),
        out_shape=jax.ShapeDtypeStruct((C1 * NP, F1), f32),
    )(y0, y0, z0, deg_col, W1b, b1r)

    zeros1 = jnp.zeros((RPT, F1), f32)
    y1 = _make_prop_kernel(e_pad, C1, F1)(z1, src2, dst2, zeros1)

    x2pre = pl.pallas_call(
        _tc2a_body,
        grid=(NB, C1),
        in_specs=[
            pl.BlockSpec((R, F1), lambda r, ci: (ci * NB + r, 0)),
            pl.BlockSpec((R, F1), lambda r, ci: (ci * NB + r, 0)),
            pl.BlockSpec((R, 1), lambda r, ci: (r, 0)),
            pl.BlockSpec((F1, C2 * F2), lambda r, ci: (ci, 0)),
        ],
        out_specs=pl.BlockSpec((R, C2 * F2), lambda r, ci: (r, 0)),
        out_shape=jax.ShapeDtypeStruct((NP, C2 * F2), f32),
        scratch_shapes=[pltpu.VMEM((R, C2 * F2), f32)],
    )(y1, z1, deg_col, W2p)

    z2 = pl.pallas_call(
        _tc2b_body,
        grid=(NB, C2),
        in_specs=[
            pl.BlockSpec((R, F2), lambda r, co: (r, co)),
            pl.BlockSpec((R, 1), lambda r, co: (r, 0)),
            pl.BlockSpec((1, 1, F2), lambda r, co: (co, 0, 0)),
        ],
        out_specs=pl.BlockSpec((R, F2), lambda r, co: (co * NB + r, 0)),
        out_shape=jax.ShapeDtypeStruct((C2 * NP, F2), f32),
    )(x2pre, deg_col, b2r)

    zeros2 = jnp.zeros((RPT, F2), f32)
    y2 = _make_prop_kernel(e_pad, C2, F2)(z2, src2, dst2, zeros2)

    dout = Wmu.shape[1]
    mu_pad, std_pad = pl.pallas_call(
        _tc3_body,
        grid=(NB, C2),
        in_specs=[
            pl.BlockSpec((R, F2), lambda r, ci: (ci * NB + r, 0)),
            pl.BlockSpec((R, F2), lambda r, ci: (ci * NB + r, 0)),
            pl.BlockSpec((R, 1), lambda r, ci: (r, 0)),
            pl.BlockSpec((F2, dout), lambda r, ci: (ci, 0)),
            pl.BlockSpec((F2, dout), lambda r, ci: (ci, 0)),
            pl.BlockSpec((1, dout), lambda r, ci: (0, 0)),
            pl.BlockSpec((1, dout), lambda r, ci: (0, 0)),
        ],
        out_specs=[
            pl.BlockSpec((R, dout), lambda r, ci: (r, 0)),
            pl.BlockSpec((R, dout), lambda r, ci: (r, 0)),
        ],
        out_shape=[
            jax.ShapeDtypeStruct((NP, dout), f32),
            jax.ShapeDtypeStruct((NP, dout), f32),
        ],
        scratch_shapes=[pltpu.VMEM((R, dout), f32), pltpu.VMEM((R, dout), f32)],
    )(y2, z2, deg_col, Wmu_p, Wstd_p, bmu.reshape(1, dout),
      bstd.reshape(1, dout))

    return (mu_pad[:N], std_pad[:N])
